# baseline (speedup 1.0000x reference)
import jax
import jax.numpy as jnp
from jax.experimental import pallas as pl
from jax.experimental.pallas import tpu as pltpu

_M = 512
_K = 8193
_N = 2049
_KM = 8192
_KS = 4
_KC = _KM // _KS
_NT = 128


def _mm_body(*refs):
    x_refs = refs[:_KS]
    w_refs = refs[_KS:2 * _KS]
    xl_ref, wl_ref, o_ref = refs[2 * _KS:]
    acc = xl_ref[...] * wl_ref[...]
    for i in range(_KS):
        wb = w_refs[i][...].astype(jnp.bfloat16)
        acc += jax.lax.dot_general(
            x_refs[i][...], wb, (((1,), (0,)), ((), ())),
            preferred_element_type=jnp.float32)
    o_ref[...] = acc


def kernel(x_affine, W):
    x_bf = x_affine.astype(jnp.bfloat16)
    x_last = x_affine[:, _KM:]
    w_last = W[_KM:, :]
    grid = (pl.cdiv(_N, _NT),)
    x_specs = [
        pl.BlockSpec((_M, _KC), lambda j, i=i: (0, i)) for i in range(_KS)
    ]
    w_specs = [
        pl.BlockSpec((_KC, _NT), lambda j, i=i: (i, j)) for i in range(_KS)
    ]
    return pl.pallas_call(
        _mm_body,
        grid=grid,
        in_specs=x_specs + w_specs + [
            pl.BlockSpec((_M, 1), lambda j: (0, 0)),
            pl.BlockSpec((1, _NT), lambda j: (0, j)),
        ],
        out_specs=pl.BlockSpec((_M, _NT), lambda j: (0, j)),
        out_shape=jax.ShapeDtypeStruct((_M, _N), jnp.float32),
        compiler_params=pltpu.CompilerParams(
            dimension_semantics=("parallel",)),
    )(*([x_bf] * _KS + [W] * _KS + [x_last, w_last]))


# speedup vs baseline: 1.1405x; 1.1405x over previous
import jax
import jax.numpy as jnp
from jax.experimental import pallas as pl
from jax.experimental.pallas import tpu as pltpu

_M = 512
_K = 8193
_N = 2049
_KM = 8192
_NT = 256


def _mm_body(x_ref, w_ref, xl_ref, wl_ref, o_ref, xs_ref):
    @pl.when(pl.program_id(0) == 0)
    def _cast_x():
        xs_ref[...] = x_ref[...].astype(jnp.bfloat16)

    wb = w_ref[...].astype(jnp.bfloat16)
    acc = jax.lax.dot_general(
        xs_ref[...], wb, (((1,), (0,)), ((), ())),
        preferred_element_type=jnp.float32)
    o_ref[...] = acc + xl_ref[...] * wl_ref[...]


def kernel(x_affine, W):
    x_last = x_affine[:, _KM:]
    w_last = W[_KM:, :]
    grid = (pl.cdiv(_N, _NT),)
    return pl.pallas_call(
        _mm_body,
        grid=grid,
        in_specs=[
            pl.BlockSpec((_M, _KM), lambda j: (0, 0)),
            pl.BlockSpec((_KM, _NT), lambda j: (0, j)),
            pl.BlockSpec((_M, 1), lambda j: (0, 0)),
            pl.BlockSpec((1, _NT), lambda j: (0, j)),
        ],
        out_specs=pl.BlockSpec((_M, _NT), lambda j: (0, j)),
        out_shape=jax.ShapeDtypeStruct((_M, _N), jnp.float32),
        scratch_shapes=[pltpu.VMEM((_M, _KM), jnp.bfloat16)],
    )(x_affine, W, x_last, w_last)


# speedup vs baseline: 1.1609x; 1.0179x over previous
import jax
import jax.numpy as jnp
from jax.experimental import pallas as pl
from jax.experimental.pallas import tpu as pltpu

_M = 512
_K = 8193
_N = 2049
_KM = 8192
_NT = 256


def _mm_body(x_ref, w_ref, o_ref, xs_ref):
    @pl.when(pl.program_id(0) == 0)
    def _cast_x():
        xs_ref[...] = x_ref[:, :_KM].astype(jnp.bfloat16)

    wb = w_ref[:_KM, :].astype(jnp.bfloat16)
    acc = jax.lax.dot_general(
        xs_ref[...], wb, (((1,), (0,)), ((), ())),
        preferred_element_type=jnp.float32)
    o_ref[...] = acc + x_ref[:, _KM:] * w_ref[_KM:, :]


def kernel(x_affine, W):
    grid = (pl.cdiv(_N, _NT),)
    return pl.pallas_call(
        _mm_body,
        grid=grid,
        in_specs=[
            pl.BlockSpec((_M, _K), lambda j: (0, 0)),
            pl.BlockSpec((_K, _NT), lambda j: (0, j)),
        ],
        out_specs=pl.BlockSpec((_M, _NT), lambda j: (0, j)),
        out_shape=jax.ShapeDtypeStruct((_M, _N), jnp.float32),
        scratch_shapes=[pltpu.VMEM((_M, _KM), jnp.bfloat16)],
    )(x_affine, W)


# speedup vs baseline: 4.0423x; 3.4821x over previous
import jax
import jax.numpy as jnp
from jax.experimental import pallas as pl
from jax.experimental.pallas import tpu as pltpu

_M = 512
_K = 8193
_N = 2049
_KM = 8192
_NT = 256


def _mm_body(wt_ref, xt_ref, o_ref, xs_ref):
    @pl.when(pl.program_id(0) == 0)
    def _cast_x():
        xs_ref[...] = xt_ref[:_KM, :].astype(jnp.bfloat16)

    wb = wt_ref[:, :_KM].astype(jnp.bfloat16)
    acc = jax.lax.dot_general(
        wb, xs_ref[...], (((1,), (0,)), ((), ())),
        preferred_element_type=jnp.float32)
    o_ref[...] = acc + wt_ref[:, _KM:] * xt_ref[_KM:, :]


def kernel(x_affine, W):
    xt = x_affine.T
    wt = W.T
    grid = (pl.cdiv(_N, _NT),)
    yt = pl.pallas_call(
        _mm_body,
        grid=grid,
        in_specs=[
            pl.BlockSpec((_NT, _K), lambda j: (j, 0)),
            pl.BlockSpec((_K, _M), lambda j: (0, 0)),
        ],
        out_specs=pl.BlockSpec((_NT, _M), lambda j: (j, 0)),
        out_shape=jax.ShapeDtypeStruct((_N, _M), jnp.float32),
        scratch_shapes=[pltpu.VMEM((_KM, _M), jnp.bfloat16)],
    )(wt, xt)
    return yt.T
